# baseline (speedup 1.0000x reference)
import jax
import jax.numpy as jnp
from jax.experimental import pallas as pl

_B = 1024
_H = 50
_K = 64
_HC = 8
_NH = (_H + _HC - 1) // _HC
_NI = 4096 // _B


def _probe_body(x_ref, out_ref):
    i = pl.program_id(1)
    a = jnp.abs(x_ref[...].reshape(_B * _HC, _K))
    s = jax.lax.dot_general(
        a, jnp.ones((_K, 128), jnp.float32), (((1,), (0,)), ((), ())),
        preferred_element_type=jnp.float32,
    )
    rz = (s == 0.0).astype(jnp.float32)

    @pl.when(i == 0)
    def _init():
        out_ref[...] = rz[None]

    @pl.when(i > 0)
    def _acc():
        out_ref[...] += rz[None]


def kernel(x, seen):
    buf_wide = pl.pallas_call(
        _probe_body,
        grid=(_NH, _NI),
        in_specs=[
            pl.BlockSpec((_B, 1, _HC, _K), lambda j, i: (i, 0, j, 0)),
        ],
        out_specs=pl.BlockSpec((1, _B * _HC, 128), lambda j, i: (j, 0, 0)),
        out_shape=jax.ShapeDtypeStruct((_NH, _B * _HC, 128), jnp.float32),
    )(x)
    buf = buf_wide[:, :, 0].reshape(_NH, _B, _HC)
    buf = buf.transpose(1, 0, 2).reshape(_B, _NH * _HC)[:, :_H]
    return (x, seen + buf)
